# Initial kernel scaffold; baseline (speedup 1.0000x reference)
import jax
import jax.numpy as jnp
from jax.experimental import pallas as pl


def kernel(output, ordinal_relation, x_A, y_A, x_B, y_B):
    raise NotImplementedError("write your pallas kernel here")


# speedup vs baseline: 2.7139x; 2.7139x over previous
import functools

import jax
import jax.numpy as jnp
from jax import lax
from jax.experimental import pallas as pl
from jax.experimental.pallas import tpu as pltpu
from jax.experimental.pallas import tpu_sc as plsc

NC, NS, L = 2, 16, 16
NW = NC * NS
B, H, W, P = 16, 512, 512, 100000
HALF = P // 2
CHUNK = 10000
NCHUNK = HALF // CHUNK
NVEC = CHUNK // L


def _softplus(s):
    v = jnp.exp(-jnp.abs(s))
    r = v / (v + 2.0)
    r2 = r * r
    poly = 1.0 + r2 * (1.0 / 3.0 + r2 * (1.0 / 5.0 + r2 * (1.0 / 7.0 + r2 * (1.0 / 9.0))))
    return jnp.maximum(s, 0.0) + 2.0 * r * poly


def _sc_body(depth_hbm, rel_hbm, xa_hbm, ya_hbm, xb_hbm, yb_hbm, out_hbm,
             bufx, bufy, idxa, idxb, bufr, za, zb, accv, sem_a, sem_b):
    wid = lax.axis_index("s") * NC + lax.axis_index("c")
    b = wid // 2
    base = b * P + (wid % 2) * HALF
    gbase = b * (H * W)

    zero = jnp.zeros((L,), jnp.float32)
    acc_log, acc_nnz, acc_sq, acc_nze = zero, zero, zero, zero

    for c in range(NCHUNK):
        off = pl.multiple_of(base + c * CHUNK, 8)
        pltpu.sync_copy(xa_hbm.at[pl.ds(off, CHUNK)], bufx)
        pltpu.sync_copy(ya_hbm.at[pl.ds(off, CHUNK)], bufy)

        def mk_idx(i, dst):
            s = pl.ds(pl.multiple_of(i * L, L), L)
            dst[s] = gbase + bufx[s] * W + bufy[s]
            return 0

        lax.fori_loop(0, NVEC, lambda i, _: mk_idx(i, idxa), 0)
        cp_a = pltpu.async_copy(depth_hbm.at[idxa], za, sem_a)
        pltpu.sync_copy(xb_hbm.at[pl.ds(off, CHUNK)], bufx)
        pltpu.sync_copy(yb_hbm.at[pl.ds(off, CHUNK)], bufy)
        lax.fori_loop(0, NVEC, lambda i, _: mk_idx(i, idxb), 0)
        cp_b = pltpu.async_copy(depth_hbm.at[idxb], zb, sem_b)
        pltpu.sync_copy(rel_hbm.at[pl.ds(off, CHUNK)], bufr)
        cp_a.wait()
        cp_b.wait()

        def acc_step(i, carry):
            a_log, a_nnz, a_sq, a_nze = carry
            s = pl.ds(pl.multiple_of(i * L, L), L)
            r = bufr[s]
            pred = za[s] - zb[s]
            t = r.astype(jnp.float32)
            nz = (r == 1) | (r == -1)
            ze = r == 0
            sp = _softplus(-t * pred)
            one = jnp.ones((L,), jnp.float32)
            a_log = a_log + jnp.where(nz, sp, 0.0)
            a_nnz = a_nnz + jnp.where(nz, one, 0.0)
            a_sq = a_sq + jnp.where(ze, pred * pred, 0.0)
            a_nze = a_nze + jnp.where(ze, one, 0.0)
            return a_log, a_nnz, a_sq, a_nze

        acc_log, acc_nnz, acc_sq, acc_nze = lax.fori_loop(
            0, NVEC, acc_step, (acc_log, acc_nnz, acc_sq, acc_nze))

    accv[0, :] = acc_log
    accv[1, :] = acc_nnz
    accv[2, :] = acc_sq
    accv[3, :] = acc_nze
    pltpu.sync_copy(accv, out_hbm.at[wid])


@functools.partial(jax.jit, static_argnames=())
def kernel(output, ordinal_relation, x_A, y_A, x_B, y_B):
    depth = output.reshape(B * H * W)
    rel = ordinal_relation.reshape(B * P)
    xa = x_A.reshape(B * P)
    ya = y_A.reshape(B * P)
    xb = x_B.reshape(B * P)
    yb = y_B.reshape(B * P)

    sc = pl.kernel(
        _sc_body,
        out_type=jax.ShapeDtypeStruct((NW, 4, L), jnp.float32),
        mesh=plsc.VectorSubcoreMesh(core_axis_name="c", subcore_axis_name="s"),
        scratch_types=[
            pltpu.VMEM((CHUNK,), jnp.int32),
            pltpu.VMEM((CHUNK,), jnp.int32),
            pltpu.VMEM((CHUNK,), jnp.int32),
            pltpu.VMEM((CHUNK,), jnp.int32),
            pltpu.VMEM((CHUNK,), jnp.int32),
            pltpu.VMEM((CHUNK,), jnp.float32),
            pltpu.VMEM((CHUNK,), jnp.float32),
            pltpu.VMEM((4, L), jnp.float32),
            pltpu.SemaphoreType.DMA,
            pltpu.SemaphoreType.DMA,
        ],
    )
    acc = sc(depth, rel, xa, ya, xb, yb)
    part = acc.sum(axis=-1).reshape(B, 2, 4).sum(axis=1)
    loss = part[:, 0] / part[:, 1] + part[:, 2] / part[:, 3]
    return jnp.sum(loss) / jnp.float32(B)


# speedup vs baseline: 3.0859x; 1.1371x over previous
import functools

import jax
import jax.numpy as jnp
from jax import lax
from jax.experimental import pallas as pl
from jax.experimental.pallas import tpu as pltpu
from jax.experimental.pallas import tpu_sc as plsc

NC, NS, L = 2, 16, 16
NW = NC * NS
B, H, W, P = 16, 512, 512, 100000
HALF = P // 2
CHUNK = 10000
NCHUNK = HALF // CHUNK
NVEC = CHUNK // L


def _softplus(s):
    v = jnp.exp(-jnp.abs(s))
    r = v / (v + 2.0)
    r2 = r * r
    poly = 1.0 + r2 * (1.0 / 3.0 + r2 * (1.0 / 5.0 + r2 * (1.0 / 7.0 + r2 * (1.0 / 9.0))))
    return jnp.maximum(s, 0.0) + 2.0 * r * poly


def _sc_body(depth_hbm, rel_hbm, xa_hbm, ya_hbm, xb_hbm, yb_hbm, out_hbm,
             bufx, bufy, idxa, idxb, bufr, za, zb, accv, sems):
    wid = lax.axis_index("s") * NC + lax.axis_index("c")
    b = wid // 2
    base = b * P + (wid % 2) * HALF
    gbase = b * (H * W)

    def stage_and_build(c, ring):
        off = pl.multiple_of(base + c * CHUNK, 8)
        pltpu.sync_copy(xa_hbm.at[pl.ds(off, CHUNK)], bufx)
        pltpu.sync_copy(ya_hbm.at[pl.ds(off, CHUNK)], bufy)

        def mk_a(i, _):
            s = pl.ds(pl.multiple_of(i * L, L), L)
            idxa[ring][s] = gbase + bufx[s] * W + bufy[s]
            return 0

        lax.fori_loop(0, NVEC, mk_a, 0)
        pltpu.sync_copy(xb_hbm.at[pl.ds(off, CHUNK)], bufx)
        pltpu.sync_copy(yb_hbm.at[pl.ds(off, CHUNK)], bufy)

        def mk_b(i, _):
            s = pl.ds(pl.multiple_of(i * L, L), L)
            idxb[ring][s] = gbase + bufx[s] * W + bufy[s]
            return 0

        lax.fori_loop(0, NVEC, mk_b, 0)
        pltpu.sync_copy(rel_hbm.at[pl.ds(off, CHUNK)], bufr[ring])

    def fire(ring):
        return (pltpu.async_copy(depth_hbm.at[idxa[ring]], za[ring], sems[2 * ring]),
                pltpu.async_copy(depth_hbm.at[idxb[ring]], zb[ring], sems[2 * ring + 1]))

    def accumulate(ring, carry):
        def acc_step(i, cr):
            a_log, a_nnz, a_sq, a_nze = cr
            s = pl.ds(pl.multiple_of(i * L, L), L)
            r = bufr[ring][s]
            pred = za[ring][s] - zb[ring][s]
            t = r.astype(jnp.float32)
            nz = (r == 1) | (r == -1)
            ze = r == 0
            sp = _softplus(-t * pred)
            one = jnp.ones((L,), jnp.float32)
            a_log = a_log + jnp.where(nz, sp, 0.0)
            a_nnz = a_nnz + jnp.where(nz, one, 0.0)
            a_sq = a_sq + jnp.where(ze, pred * pred, 0.0)
            a_nze = a_nze + jnp.where(ze, one, 0.0)
            return a_log, a_nnz, a_sq, a_nze

        return lax.fori_loop(0, NVEC, acc_step, carry)

    zero = jnp.zeros((L,), jnp.float32)
    carry = (zero, zero, zero, zero)

    stage_and_build(0, 0)
    inflight = fire(0)
    for c in range(NCHUNK):
        ring, nring = c % 2, (c + 1) % 2
        if c + 1 < NCHUNK:
            stage_and_build(c + 1, nring)
        for cp in inflight:
            cp.wait()
        if c + 1 < NCHUNK:
            inflight = fire(nring)
        carry = accumulate(ring, carry)

    acc_log, acc_nnz, acc_sq, acc_nze = carry
    accv[0, :] = acc_log
    accv[1, :] = acc_nnz
    accv[2, :] = acc_sq
    accv[3, :] = acc_nze
    pltpu.sync_copy(accv, out_hbm.at[wid])


@functools.partial(jax.jit, static_argnames=())
def kernel(output, ordinal_relation, x_A, y_A, x_B, y_B):
    depth = output.reshape(B * H * W)
    rel = ordinal_relation.reshape(B * P)
    xa = x_A.reshape(B * P)
    ya = y_A.reshape(B * P)
    xb = x_B.reshape(B * P)
    yb = y_B.reshape(B * P)

    sc = pl.kernel(
        _sc_body,
        out_type=jax.ShapeDtypeStruct((NW, 4, L), jnp.float32),
        mesh=plsc.VectorSubcoreMesh(core_axis_name="c", subcore_axis_name="s"),
        scratch_types=[
            pltpu.VMEM((CHUNK,), jnp.int32),
            pltpu.VMEM((CHUNK,), jnp.int32),
            [pltpu.VMEM((CHUNK,), jnp.int32)] * 2,
            [pltpu.VMEM((CHUNK,), jnp.int32)] * 2,
            [pltpu.VMEM((CHUNK,), jnp.int32)] * 2,
            [pltpu.VMEM((CHUNK,), jnp.float32)] * 2,
            [pltpu.VMEM((CHUNK,), jnp.float32)] * 2,
            pltpu.VMEM((4, L), jnp.float32),
            [pltpu.SemaphoreType.DMA] * 4,
        ],
    )
    acc = sc(depth, rel, xa, ya, xb, yb)
    part = acc.sum(axis=-1).reshape(B, 2, 4).sum(axis=1)
    loss = part[:, 0] / part[:, 1] + part[:, 2] / part[:, 3]
    return jnp.sum(loss) / jnp.float32(B)
